# baseline (speedup 1.0000x reference)
import functools

import jax
import jax.numpy as jnp
from jax import lax
from jax.experimental import pallas as pl
from jax.experimental.pallas import tpu as pltpu
from jax.experimental.pallas import tpu_sc as plsc

N_NODES = 10000
N_EDGES = 160000
D_HID = 64

NC = 2
NS = 16
NW = NC * NS
CHUNK = 128
CH_PER_W = 40
GRP = 4
NGRP = CH_PER_W // GRP
NPAIR = NGRP // 2
E_PAD = NW * CH_PER_W * CHUNK
N_CHUNKS = E_PAD // CHUNK
ACC_ROWS = N_NODES + 16

ROWS_PER_SUB = 624
ROWS_REM = N_NODES - NS * ROWS_PER_SUB
REM_ROW0 = NS * ROWS_PER_SUB

N_PACK = N_NODES // 2
ROW_BLK = 1000
N_ROW_BLKS = N_PACK // ROW_BLK


def _segsum_sc(y, src2d, dst2d, zeros):
    mesh = plsc.VectorSubcoreMesh(core_axis_name="c", subcore_axis_name="s")

    @functools.partial(
        pl.kernel,
        mesh=mesh,
        compiler_params=pltpu.CompilerParams(use_tc_tiling_on_sc=False),
        out_type=jax.ShapeDtypeStruct((NC, N_NODES, D_HID), jnp.float32),
        scratch_types=[
            pltpu.VMEM((CH_PER_W, CHUNK), jnp.int32),
            pltpu.VMEM((CH_PER_W, CHUNK), jnp.int32),
            pltpu.VMEM((GRP * CHUNK, D_HID), jnp.float32),
            pltpu.VMEM((GRP * CHUNK, D_HID), jnp.float32),
            pltpu.VMEM_SHARED((ACC_ROWS, D_HID), jnp.float32),
            pltpu.SemaphoreType.DMA,
            pltpu.SemaphoreType.DMA,
        ],
    )
    def seg(y_hbm, src_hbm, dst_hbm, zeros_hbm, out_hbm,
            srcv, dstv, bankA, bankB, acc, gsem, ssem):
        cid = lax.axis_index("c")
        sid = lax.axis_index("s")
        wid = cid * NS + sid
        chunk0 = wid * CH_PER_W

        pltpu.sync_copy(src_hbm.at[pl.ds(chunk0, CH_PER_W)], srcv)
        pltpu.sync_copy(dst_hbm.at[pl.ds(chunk0, CH_PER_W)], dstv)

        row0 = pl.multiple_of(sid * ROWS_PER_SUB, 8)
        pltpu.sync_copy(zeros_hbm.at[pl.ds(row0, ROWS_PER_SUB)],
                        acc.at[pl.ds(row0, ROWS_PER_SUB)])

        @pl.when(sid == NS - 1)
        def _zero_rem():
            pltpu.sync_copy(zeros_hbm.at[pl.ds(REM_ROW0, ROWS_REM)],
                            acc.at[pl.ds(REM_ROW0, ROWS_REM)])

        plsc.subcore_barrier()

        def fire_g(bank, grp):
            for j in range(GRP):
                pltpu.async_copy(y_hbm.at[srcv.at[grp * GRP + j]],
                                 bank.at[pl.ds(j * CHUNK, CHUNK)], gsem)

        def drain_g(bank, grp):
            for j in range(GRP):
                pltpu.make_async_copy(
                    y_hbm.at[srcv.at[grp * GRP + j]],
                    bank.at[pl.ds(j * CHUNK, CHUNK)], gsem).wait()

        def fire_s(bank, grp):
            for j in range(GRP):
                pltpu.async_copy(bank.at[pl.ds(j * CHUNK, CHUNK)],
                                 acc.at[dstv.at[grp * GRP + j]], ssem,
                                 add=True)

        def drain_s(bank, grp):
            for j in range(GRP):
                pltpu.make_async_copy(
                    bank.at[pl.ds(j * CHUNK, CHUNK)],
                    acc.at[dstv.at[grp * GRP + j]], ssem).wait()

        fire_g(bankA, 0)

        def pair(k, _):
            gA = k * 2
            drain_g(bankA, gA)
            fire_g(bankB, gA + 1)
            fire_s(bankA, gA)
            drain_s(bankA, gA)
            drain_g(bankB, gA + 1)

            @pl.when(k < NPAIR - 1)
            def _next():
                fire_g(bankA, gA + 2)

            fire_s(bankB, gA + 1)
            drain_s(bankB, gA + 1)
            return ()

        lax.fori_loop(0, NPAIR, pair, ())

        plsc.subcore_barrier()
        pltpu.sync_copy(acc.at[pl.ds(row0, ROWS_PER_SUB)],
                        out_hbm.at[cid, pl.ds(row0, ROWS_PER_SUB)])

        @pl.when(sid == NS - 1)
        def _out_rem():
            pltpu.sync_copy(acc.at[pl.ds(REM_ROW0, ROWS_REM)],
                            out_hbm.at[cid, pl.ds(REM_ROW0, ROWS_REM)])

    return seg(y, src2d, dst2d, zeros)


def _full(shape):
    return pl.BlockSpec(shape, lambda i: tuple(0 for _ in shape))


def _rows(width):
    return pl.BlockSpec((ROW_BLK, width), lambda i: (i, 0))


def _prows(width):
    return pl.BlockSpec((NC, ROW_BLK, width), lambda i: (0, i, 0))


def _mm2_body(xa_ref, xb_ref, wa_ref, wb_ref, b_ref, ya_ref, yb_ref):
    xa = xa_ref[...]
    xb = xb_ref[...]
    wa = wa_ref[...]
    wb = wb_ref[...]
    ya_ref[...] = jnp.concatenate(
        [jnp.dot(xa, wa, preferred_element_type=jnp.float32),
         jnp.dot(xb, wa, preferred_element_type=jnp.float32)], axis=1)
    yb_ref[...] = jnp.concatenate(
        [jnp.dot(xa, wb, preferred_element_type=jnp.float32),
         jnp.dot(xb, wb, preferred_element_type=jnp.float32)],
        axis=1) + b_ref[...]


def _stage_in(x, wa, wb, b2x, width_in, width_out):
    xlo = pl.BlockSpec((ROW_BLK, width_in), lambda i: (i, 0))
    xhi = pl.BlockSpec((ROW_BLK, width_in), lambda i: (i + N_ROW_BLKS, 0))
    return pl.pallas_call(
        _mm2_body,
        grid=(N_ROW_BLKS,),
        in_specs=[xlo, xhi, _full(wa.shape), _full(wb.shape),
                  _full((1, width_out))],
        out_specs=[_rows(width_out), _rows(width_out)],
        out_shape=[jax.ShapeDtypeStruct((N_PACK, width_out), jnp.float32),
                   jax.ShapeDtypeStruct((N_PACK, width_out), jnp.float32)],
    )(x, x, wa, wb, b2x.reshape(1, -1))


def _combine_body(p_ref, r_ref, wa_ref, wb_ref, b_ref, ya_ref, yb_ref):
    h = jnp.maximum(p_ref[0] + p_ref[1] + r_ref[...], 0.0)
    ya_ref[...] = jnp.dot(h, wa_ref[...], preferred_element_type=jnp.float32)
    yb_ref[...] = (jnp.dot(h, wb_ref[...], preferred_element_type=jnp.float32)
                   + b_ref[...])


def _stage_mid(p, r, wa, wb, b, width_out):
    return pl.pallas_call(
        _combine_body,
        grid=(N_ROW_BLKS,),
        in_specs=[_prows(2 * D_HID), _rows(2 * D_HID),
                  _full(wa.shape), _full(wb.shape), _full((1, width_out))],
        out_specs=[_rows(width_out), _rows(width_out)],
        out_shape=[jax.ShapeDtypeStruct((N_PACK, width_out), jnp.float32),
                   jax.ShapeDtypeStruct((N_PACK, width_out), jnp.float32)],
    )(p, r, wa, wb, b.reshape(1, -1))


def _combine_id_body(p_ref, r_ref, wb_ref, b_ref, h_ref, rb_ref):
    h = jnp.maximum(p_ref[0] + p_ref[1] + r_ref[...], 0.0)
    h_ref[...] = h
    rb_ref[...] = (jnp.dot(h, wb_ref[...], preferred_element_type=jnp.float32)
                   + b_ref[...])


def _stage_mid_id(p, r, wb, b, width_out):
    return pl.pallas_call(
        _combine_id_body,
        grid=(N_ROW_BLKS,),
        in_specs=[_prows(2 * D_HID), _rows(2 * D_HID),
                  _full(wb.shape), _full((1, width_out))],
        out_specs=[_rows(2 * D_HID), _rows(width_out)],
        out_shape=[jax.ShapeDtypeStruct((N_PACK, 2 * D_HID), jnp.float32),
                   jax.ShapeDtypeStruct((N_PACK, width_out), jnp.float32)],
    )(p, r, wb, b.reshape(1, -1))


def _head_body(p_ref, r3_ref, wrel3_ref, fc1w_ref, fc1b_ref,
               fc2w_ref, fc2b_ref, out_ref):
    agg = p_ref[0] + p_ref[1]
    z = jnp.dot(agg, wrel3_ref[...], preferred_element_type=jnp.float32)
    z = jnp.maximum(z + r3_ref[...], 0.0)
    z = jnp.dot(z, fc1w_ref[...], preferred_element_type=jnp.float32)
    z = jnp.maximum(z + fc1b_ref[...], 0.0)
    logits = (jnp.dot(z, fc2w_ref[...], preferred_element_type=jnp.float32)
              + fc2b_ref[...])
    n = logits.shape[-1] // 2
    out = []
    for l in (logits[:, :n], logits[:, n:]):
        m = jnp.max(l, axis=-1, keepdims=True)
        lse = jnp.log(jnp.sum(jnp.exp(l - m), axis=-1, keepdims=True)) + m
        out.append(l - lse)
    out_ref[...] = jnp.concatenate(out, axis=-1)


def _stage_head(p, r3, wrel3, fc1w, fc1b, fc2w, fc2b):
    n_cls2 = fc2w.shape[1]
    return pl.pallas_call(
        _head_body,
        grid=(N_ROW_BLKS,),
        in_specs=[_prows(2 * D_HID), _rows(256),
                  _full(wrel3.shape), _full(fc1w.shape), _full((1, 256)),
                  _full(fc2w.shape), _full((1, n_cls2))],
        out_specs=[_rows(n_cls2)],
        out_shape=[jax.ShapeDtypeStruct((N_PACK, n_cls2), jnp.float32)],
    )(p, r3, wrel3, fc1w, fc1b.reshape(1, -1),
      fc2w, fc2b.reshape(1, -1))[0]


def _bd(w):
    z = jnp.zeros(w.shape, w.dtype)
    return jnp.concatenate(
        [jnp.concatenate([w, z], axis=1), jnp.concatenate([z, w], axis=1)],
        axis=0)


def kernel(x, edge_index, W_rel1, W_root1, b1, W_rel2, W_root2, b2,
           W_rel3, W_root3, b3, fc1_W, fc1_b, fc2_W, fc2_b):
    def perm(i):
        return 2 * (i % N_PACK) + i // N_PACK

    src = perm(edge_index[0].astype(jnp.int32))
    dst = perm(edge_index[1].astype(jnp.int32))
    pad = E_PAD - N_EDGES
    pad_iota = lax.iota(jnp.int32, pad)
    src2d = jnp.concatenate(
        [src, pad_iota % N_NODES]).reshape(N_CHUNKS, CHUNK)
    dst2d = jnp.concatenate(
        [dst, N_NODES + pad_iota % 16]).reshape(N_CHUNKS, CHUNK)
    zeros = jnp.zeros((N_NODES, D_HID), jnp.float32)

    b2p = jnp.concatenate([b2, b2])
    b3p = jnp.concatenate([b3, b3])

    y1, r1 = _stage_in(x, W_rel1, W_root1,
                       jnp.concatenate([b1, b1]), 256, 128)
    p1 = _segsum_sc(y1.reshape(N_NODES, D_HID), src2d, dst2d, zeros)

    y2, r2 = _stage_mid(p1.reshape(NC, N_PACK, 2 * D_HID), r1,
                        _bd(W_rel2), _bd(W_root2), b2p, 128)
    p2 = _segsum_sc(y2.reshape(N_NODES, D_HID), src2d, dst2d, zeros)

    h2, r3 = _stage_mid_id(p2.reshape(NC, N_PACK, 2 * D_HID), r2,
                           _bd(W_root3), b3p, 256)
    p3 = _segsum_sc(h2.reshape(N_NODES, D_HID), src2d, dst2d, zeros)

    outp = _stage_head(p3.reshape(NC, N_PACK, 2 * D_HID), r3, _bd(W_rel3),
                       _bd(fc1_W), jnp.concatenate([fc1_b, fc1_b]),
                       _bd(fc2_W), jnp.concatenate([fc2_b, fc2_b]))
    n_cls = fc2_W.shape[1]
    return jnp.concatenate([outp[:, :n_cls], outp[:, n_cls:]], axis=0)


# speedup vs baseline: 1.0234x; 1.0234x over previous
import functools

import jax
import jax.numpy as jnp
from jax import lax
from jax.experimental import pallas as pl
from jax.experimental.pallas import tpu as pltpu
from jax.experimental.pallas import tpu_sc as plsc

N_NODES = 10000
N_EDGES = 160000
D_HID = 64

NC = 2
NS = 16
NW = NC * NS
CHUNK = 128
CH_PER_W = 40
GRP = 4
NGRP = CH_PER_W // GRP
NPAIR = NGRP // 2
E_PAD = NW * CH_PER_W * CHUNK
N_CHUNKS = E_PAD // CHUNK
ACC_ROWS = N_NODES + 16

ROWS_PER_SUB = 624
ROWS_REM = N_NODES - NS * ROWS_PER_SUB
REM_ROW0 = NS * ROWS_PER_SUB

N_PACK = N_NODES // 2
ROW_BLK = 1000
N_ROW_BLKS = N_PACK // ROW_BLK


def _segsum_sc(y, src2d, dst2d, zeros):
    mesh = plsc.VectorSubcoreMesh(core_axis_name="c", subcore_axis_name="s")

    @functools.partial(
        pl.kernel,
        mesh=mesh,
        compiler_params=pltpu.CompilerParams(use_tc_tiling_on_sc=False),
        out_type=jax.ShapeDtypeStruct((NC, N_NODES, D_HID), jnp.float32),
        scratch_types=[
            pltpu.VMEM((CH_PER_W, CHUNK), jnp.int32),
            pltpu.VMEM((CH_PER_W, CHUNK), jnp.int32),
            pltpu.VMEM((GRP * CHUNK, D_HID), jnp.float32),
            pltpu.VMEM((GRP * CHUNK, D_HID), jnp.float32),
            pltpu.VMEM_SHARED((ACC_ROWS, D_HID), jnp.float32),
            pltpu.SemaphoreType.DMA,
            pltpu.SemaphoreType.DMA,
        ],
    )
    def seg(y_hbm, src_hbm, dst_hbm, zeros_hbm, out_hbm,
            srcv, dstv, bankA, bankB, acc, gsem, ssem):
        cid = lax.axis_index("c")
        sid = lax.axis_index("s")
        wid = cid * NS + sid
        chunk0 = wid * CH_PER_W

        pltpu.sync_copy(src_hbm.at[pl.ds(chunk0, CH_PER_W)], srcv)
        pltpu.sync_copy(dst_hbm.at[pl.ds(chunk0, CH_PER_W)], dstv)

        row0 = pl.multiple_of(sid * ROWS_PER_SUB, 8)
        pltpu.sync_copy(zeros_hbm.at[pl.ds(row0, ROWS_PER_SUB)],
                        acc.at[pl.ds(row0, ROWS_PER_SUB)])

        @pl.when(sid == NS - 1)
        def _zero_rem():
            pltpu.sync_copy(zeros_hbm.at[pl.ds(REM_ROW0, ROWS_REM)],
                            acc.at[pl.ds(REM_ROW0, ROWS_REM)])

        plsc.subcore_barrier()

        def fire_g(bank, grp):
            for j in range(GRP):
                pltpu.async_copy(y_hbm.at[srcv.at[grp * GRP + j]],
                                 bank.at[pl.ds(j * CHUNK, CHUNK)], gsem)

        def drain_g(bank, grp):
            for j in range(GRP):
                pltpu.make_async_copy(
                    y_hbm.at[srcv.at[grp * GRP + j]],
                    bank.at[pl.ds(j * CHUNK, CHUNK)], gsem).wait()

        def fire_s(bank, grp):
            for j in range(GRP):
                pltpu.async_copy(bank.at[pl.ds(j * CHUNK, CHUNK)],
                                 acc.at[dstv.at[grp * GRP + j]], ssem,
                                 add=True)

        def drain_s(bank, grp):
            for j in range(GRP):
                pltpu.make_async_copy(
                    bank.at[pl.ds(j * CHUNK, CHUNK)],
                    acc.at[dstv.at[grp * GRP + j]], ssem).wait()

        fire_g(bankA, 0)

        def pair(k, _):
            gA = k * 2
            drain_g(bankA, gA)
            fire_g(bankB, gA + 1)
            fire_s(bankA, gA)
            drain_s(bankA, gA)
            drain_g(bankB, gA + 1)

            @pl.when(k < NPAIR - 1)
            def _next():
                fire_g(bankA, gA + 2)

            fire_s(bankB, gA + 1)
            drain_s(bankB, gA + 1)
            return ()

        lax.fori_loop(0, NPAIR, pair, ())

        plsc.subcore_barrier()
        pltpu.sync_copy(acc.at[pl.ds(row0, ROWS_PER_SUB)],
                        out_hbm.at[cid, pl.ds(row0, ROWS_PER_SUB)])

        @pl.when(sid == NS - 1)
        def _out_rem():
            pltpu.sync_copy(acc.at[pl.ds(REM_ROW0, ROWS_REM)],
                            out_hbm.at[cid, pl.ds(REM_ROW0, ROWS_REM)])

    return seg(y, src2d, dst2d, zeros)


def _full(shape):
    return pl.BlockSpec(shape, lambda i: tuple(0 for _ in shape))


def _rows(width):
    return pl.BlockSpec((ROW_BLK, width), lambda i: (i, 0))


def _prows(width):
    return pl.BlockSpec((NC, ROW_BLK, width), lambda i: (0, i, 0))


def _mm2_body(xa_ref, xb_ref, wa_ref, wb_ref, b_ref, ya_ref, yb_ref):
    xa = xa_ref[...]
    xb = xb_ref[...]
    wa = wa_ref[...]
    wb = wb_ref[...]
    ya_ref[...] = jnp.concatenate(
        [jnp.dot(xa, wa, preferred_element_type=jnp.float32),
         jnp.dot(xb, wa, preferred_element_type=jnp.float32)], axis=1)
    yb_ref[...] = jnp.concatenate(
        [jnp.dot(xa, wb, preferred_element_type=jnp.float32),
         jnp.dot(xb, wb, preferred_element_type=jnp.float32)],
        axis=1) + b_ref[...]


def _stage_in(x, wa, wb, b2x, width_in, width_out):
    xlo = pl.BlockSpec((ROW_BLK, width_in), lambda i: (i, 0))
    xhi = pl.BlockSpec((ROW_BLK, width_in), lambda i: (i + N_ROW_BLKS, 0))
    return pl.pallas_call(
        _mm2_body,
        grid=(N_ROW_BLKS,),
        in_specs=[xlo, xhi, _full(wa.shape), _full(wb.shape),
                  _full((1, width_out))],
        out_specs=[_rows(width_out), _rows(width_out)],
        out_shape=[jax.ShapeDtypeStruct((N_PACK, width_out), jnp.float32),
                   jax.ShapeDtypeStruct((N_PACK, width_out), jnp.float32)],
    )(x, x, wa, wb, b2x.reshape(1, -1))


def _combine_body(p_ref, r_ref, wa_ref, wb_ref, b_ref, ya_ref, yb_ref):
    h = jnp.maximum(p_ref[0] + p_ref[1] + r_ref[...], 0.0)
    ya_ref[...] = jnp.dot(h, wa_ref[...], preferred_element_type=jnp.float32)
    yb_ref[...] = (jnp.dot(h, wb_ref[...], preferred_element_type=jnp.float32)
                   + b_ref[...])


def _stage_mid(p, r, wa, wb, b, width_out):
    return pl.pallas_call(
        _combine_body,
        grid=(N_ROW_BLKS,),
        in_specs=[_prows(2 * D_HID), _rows(2 * D_HID),
                  _full(wa.shape), _full(wb.shape), _full((1, width_out))],
        out_specs=[_rows(width_out), _rows(width_out)],
        out_shape=[jax.ShapeDtypeStruct((N_PACK, width_out), jnp.float32),
                   jax.ShapeDtypeStruct((N_PACK, width_out), jnp.float32)],
    )(p, r, wa, wb, b.reshape(1, -1))


def _combine_id_body(p_ref, r_ref, wb_ref, b_ref, h_ref, rb_ref):
    h = jnp.maximum(p_ref[0] + p_ref[1] + r_ref[...], 0.0)
    h_ref[...] = h
    rb_ref[...] = (jnp.dot(h, wb_ref[...], preferred_element_type=jnp.float32)
                   + b_ref[...])


def _stage_mid_id(p, r, wb, b, width_out):
    return pl.pallas_call(
        _combine_id_body,
        grid=(N_ROW_BLKS,),
        in_specs=[_prows(2 * D_HID), _rows(2 * D_HID),
                  _full(wb.shape), _full((1, width_out))],
        out_specs=[_rows(2 * D_HID), _rows(width_out)],
        out_shape=[jax.ShapeDtypeStruct((N_PACK, 2 * D_HID), jnp.float32),
                   jax.ShapeDtypeStruct((N_PACK, width_out), jnp.float32)],
    )(p, r, wb, b.reshape(1, -1))


def _head_body(p_ref, r3_ref, wrel3_ref, fc1w_ref, fc1b_ref,
               fc2w_ref, fc2b_ref, out_ref):
    agg = p_ref[0] + p_ref[1]
    z = jnp.dot(agg, wrel3_ref[...], preferred_element_type=jnp.float32)
    z = jnp.maximum(z + r3_ref[...], 0.0)
    z = jnp.dot(z, fc1w_ref[...], preferred_element_type=jnp.float32)
    z = jnp.maximum(z + fc1b_ref[...], 0.0)
    logits = (jnp.dot(z, fc2w_ref[...], preferred_element_type=jnp.float32)
              + fc2b_ref[...])
    n = logits.shape[-1] // 2
    out = []
    for l in (logits[:, :n], logits[:, n:]):
        m = jnp.max(l, axis=-1, keepdims=True)
        lse = jnp.log(jnp.sum(jnp.exp(l - m), axis=-1, keepdims=True)) + m
        out.append(l - lse)
    out_ref[...] = jnp.concatenate(out, axis=-1)


def _stage_head(p, r3, wrel3, fc1w, fc1b, fc2w, fc2b):
    n_cls2 = fc2w.shape[1]
    return pl.pallas_call(
        _head_body,
        grid=(N_ROW_BLKS,),
        in_specs=[_prows(2 * D_HID), _rows(256),
                  _full(wrel3.shape), _full(fc1w.shape), _full((1, 256)),
                  _full(fc2w.shape), _full((1, n_cls2))],
        out_specs=[_rows(n_cls2)],
        out_shape=[jax.ShapeDtypeStruct((N_PACK, n_cls2), jnp.float32)],
    )(p, r3, wrel3, fc1w, fc1b.reshape(1, -1),
      fc2w, fc2b.reshape(1, -1))[0]


def _bd(w):
    z = jnp.zeros(w.shape, w.dtype)
    return jnp.concatenate(
        [jnp.concatenate([w, z], axis=1), jnp.concatenate([z, w], axis=1)],
        axis=0)


def kernel(x, edge_index, W_rel1, W_root1, b1, W_rel2, W_root2, b2,
           W_rel3, W_root3, b3, fc1_W, fc1_b, fc2_W, fc2_b):
    def perm(i):
        return jnp.where(i < N_PACK, 2 * i, 2 * i - (2 * N_PACK - 1))

    src = perm(edge_index[0].astype(jnp.int32))
    dst = perm(edge_index[1].astype(jnp.int32))
    pad = E_PAD - N_EDGES
    pad_iota = lax.iota(jnp.int32, pad)
    src2d = jnp.concatenate(
        [src, pad_iota]).reshape(N_CHUNKS, CHUNK)
    dst2d = jnp.concatenate(
        [dst, N_NODES + pad_iota % 16]).reshape(N_CHUNKS, CHUNK)
    zeros = jnp.zeros((N_NODES, D_HID), jnp.float32)

    b2p = jnp.concatenate([b2, b2])
    b3p = jnp.concatenate([b3, b3])

    y1, r1 = _stage_in(x, W_rel1, W_root1,
                       jnp.concatenate([b1, b1]), 256, 128)
    p1 = _segsum_sc(y1.reshape(N_NODES, D_HID), src2d, dst2d, zeros)

    y2, r2 = _stage_mid(p1.reshape(NC, N_PACK, 2 * D_HID), r1,
                        _bd(W_rel2), _bd(W_root2), b2p, 128)
    p2 = _segsum_sc(y2.reshape(N_NODES, D_HID), src2d, dst2d, zeros)

    h2, r3 = _stage_mid_id(p2.reshape(NC, N_PACK, 2 * D_HID), r2,
                           _bd(W_root3), b3p, 256)
    p3 = _segsum_sc(h2.reshape(N_NODES, D_HID), src2d, dst2d, zeros)

    outp = _stage_head(p3.reshape(NC, N_PACK, 2 * D_HID), r3, _bd(W_rel3),
                       _bd(fc1_W), jnp.concatenate([fc1_b, fc1_b]),
                       _bd(fc2_W), jnp.concatenate([fc2_b, fc2_b]))
    n_cls = fc2_W.shape[1]
    return jnp.concatenate([outp[:, :n_cls], outp[:, n_cls:]], axis=0)


# speedup vs baseline: 1.0503x; 1.0263x over previous
import functools

import jax
import jax.numpy as jnp
from jax import lax
from jax.experimental import pallas as pl
from jax.experimental.pallas import tpu as pltpu
from jax.experimental.pallas import tpu_sc as plsc

N_NODES = 10000
N_EDGES = 160000
D_HID = 64

NC = 2
NS = 16
NW = NC * NS
CHUNK = 128
N_CHUNKS = N_EDGES // CHUNK
CH_PER_W = 40
GRP = 5
LAST_W = NW - 1
LAST_CH = N_CHUNKS - LAST_W * CH_PER_W
ACC_ROWS = N_NODES

ROWS_PER_SUB = 624
ROWS_REM = N_NODES - NS * ROWS_PER_SUB
REM_ROW0 = NS * ROWS_PER_SUB

N_PACK = N_NODES // 2
ROW_BLK = 1000
N_ROW_BLKS = N_PACK // ROW_BLK


def _segsum_sc(y, src2d, dst2d, zeros):
    mesh = plsc.VectorSubcoreMesh(core_axis_name="c", subcore_axis_name="s")

    @functools.partial(
        pl.kernel,
        mesh=mesh,
        compiler_params=pltpu.CompilerParams(use_tc_tiling_on_sc=False),
        out_type=jax.ShapeDtypeStruct((NC, N_NODES, D_HID), jnp.float32),
        scratch_types=[
            pltpu.VMEM((CH_PER_W, CHUNK), jnp.int32),
            pltpu.VMEM((2, GRP, CHUNK), jnp.int32),
            pltpu.VMEM((GRP * CHUNK, D_HID), jnp.float32),
            pltpu.VMEM((GRP * CHUNK, D_HID), jnp.float32),
            pltpu.VMEM_SHARED((ACC_ROWS, D_HID), jnp.float32),
            pltpu.SemaphoreType.DMA,
            pltpu.SemaphoreType.DMA,
            pltpu.SemaphoreType.DMA,
        ],
    )
    def seg(y_hbm, src_hbm, dst_hbm, zeros_hbm, out_hbm,
            srcv, dstv, bankA, bankB, acc, gsem, ssem, isem):
        cid = lax.axis_index("c")
        sid = lax.axis_index("s")
        wid = cid * NS + sid
        chunk0 = wid * CH_PER_W
        n_ch = jnp.where(wid == LAST_W, LAST_CH, CH_PER_W)

        def _perm(v):
            return jnp.where(v < N_PACK, 2 * v, 2 * v - (2 * N_PACK - 1))

        pltpu.sync_copy(src_hbm.at[pl.ds(chunk0, LAST_CH)],
                        srcv.at[pl.ds(0, LAST_CH)])

        @pl.when(wid < LAST_W)
        def _load_rest():
            pltpu.sync_copy(src_hbm.at[pl.ds(chunk0 + LAST_CH,
                                             CH_PER_W - LAST_CH)],
                            srcv.at[pl.ds(LAST_CH, CH_PER_W - LAST_CH)])

        def permute_src(i, _):
            r = lax.shift_right_logical(i, 3)
            c = lax.shift_left(jnp.bitwise_and(i, 7), 4)
            srcv[r, pl.ds(c, 16)] = _perm(srcv[r, pl.ds(c, 16)])
            return ()

        lax.fori_loop(0, n_ch * (CHUNK // 16), permute_src, ())

        def fire_i(db, grp):
            pltpu.async_copy(dst_hbm.at[pl.ds(chunk0 + grp * GRP, GRP)],
                             dstv.at[db], isem)

        def drain_i(db, grp):
            pltpu.make_async_copy(
                dst_hbm.at[pl.ds(chunk0 + grp * GRP, GRP)],
                dstv.at[db], isem).wait()

        def perm_d(db):
            for r in range(GRP):
                for c in range(CHUNK // 16):
                    sl = pl.ds(c * 16, 16)
                    dstv[db, r, sl] = _perm(dstv[db, r, sl])

        row0 = pl.multiple_of(sid * ROWS_PER_SUB, 8)
        pltpu.sync_copy(zeros_hbm.at[pl.ds(row0, ROWS_PER_SUB)],
                        acc.at[pl.ds(row0, ROWS_PER_SUB)])

        @pl.when(sid == NS - 1)
        def _zero_rem():
            pltpu.sync_copy(zeros_hbm.at[pl.ds(REM_ROW0, ROWS_REM)],
                            acc.at[pl.ds(REM_ROW0, ROWS_REM)])

        plsc.subcore_barrier()

        def fire_g(bank, grp):
            for j in range(GRP):
                pltpu.async_copy(y_hbm.at[srcv.at[grp * GRP + j]],
                                 bank.at[pl.ds(j * CHUNK, CHUNK)], gsem)

        def drain_g(bank, grp):
            for j in range(GRP):
                pltpu.make_async_copy(
                    y_hbm.at[srcv.at[grp * GRP + j]],
                    bank.at[pl.ds(j * CHUNK, CHUNK)], gsem).wait()

        def fire_s(bank, db):
            for j in range(GRP):
                pltpu.async_copy(bank.at[pl.ds(j * CHUNK, CHUNK)],
                                 acc.at[dstv.at[db, j]], ssem, add=True)

        def drain_s(bank, db):
            for j in range(GRP):
                pltpu.make_async_copy(
                    bank.at[pl.ds(j * CHUNK, CHUNK)],
                    acc.at[dstv.at[db, j]], ssem).wait()

        fire_i(0, 0)
        fire_g(bankA, 0)
        npair = n_ch // (2 * GRP)

        def pair(k, _):
            gA = k * 2
            drain_i(0, gA)
            perm_d(0)
            fire_i(1, gA + 1)
            drain_g(bankA, gA)
            fire_g(bankB, gA + 1)
            fire_s(bankA, 0)
            drain_s(bankA, 0)
            drain_i(1, gA + 1)
            perm_d(1)
            drain_g(bankB, gA + 1)

            @pl.when(k < npair - 1)
            def _next():
                fire_g(bankA, gA + 2)
                fire_i(0, gA + 2)

            fire_s(bankB, 1)
            drain_s(bankB, 1)
            return ()

        lax.fori_loop(0, npair, pair, ())

        plsc.subcore_barrier()
        pltpu.sync_copy(acc.at[pl.ds(row0, ROWS_PER_SUB)],
                        out_hbm.at[cid, pl.ds(row0, ROWS_PER_SUB)])

        @pl.when(sid == NS - 1)
        def _out_rem():
            pltpu.sync_copy(acc.at[pl.ds(REM_ROW0, ROWS_REM)],
                            out_hbm.at[cid, pl.ds(REM_ROW0, ROWS_REM)])

    return seg(y, src2d, dst2d, zeros)


def _full(shape):
    return pl.BlockSpec(shape, lambda i: tuple(0 for _ in shape))


def _rows(width):
    return pl.BlockSpec((ROW_BLK, width), lambda i: (i, 0))


def _prows(width):
    return pl.BlockSpec((NC, ROW_BLK, width), lambda i: (0, i, 0))


def _mm2_body(xa_ref, xb_ref, wa_ref, wb_ref, b_ref, ya_ref, yb_ref):
    xa = xa_ref[...]
    xb = xb_ref[...]
    wa = wa_ref[...]
    wb = wb_ref[...]
    ya_ref[...] = jnp.concatenate(
        [jnp.dot(xa, wa, preferred_element_type=jnp.float32),
         jnp.dot(xb, wa, preferred_element_type=jnp.float32)], axis=1)
    yb_ref[...] = jnp.concatenate(
        [jnp.dot(xa, wb, preferred_element_type=jnp.float32),
         jnp.dot(xb, wb, preferred_element_type=jnp.float32)],
        axis=1) + b_ref[...]


def _stage_in(x, wa, wb, b2x, width_in, width_out):
    xlo = pl.BlockSpec((ROW_BLK, width_in), lambda i: (i, 0))
    xhi = pl.BlockSpec((ROW_BLK, width_in), lambda i: (i + N_ROW_BLKS, 0))
    return pl.pallas_call(
        _mm2_body,
        grid=(N_ROW_BLKS,),
        in_specs=[xlo, xhi, _full(wa.shape), _full(wb.shape),
                  _full((1, width_out))],
        out_specs=[_rows(width_out), _rows(width_out)],
        out_shape=[jax.ShapeDtypeStruct((N_PACK, width_out), jnp.float32),
                   jax.ShapeDtypeStruct((N_PACK, width_out), jnp.float32)],
    )(x, x, wa, wb, b2x.reshape(1, -1))


def _combine_body(p_ref, r_ref, wa_ref, wb_ref, b_ref, ya_ref, yb_ref):
    h = jnp.maximum(p_ref[0] + p_ref[1] + r_ref[...], 0.0)
    ya_ref[...] = jnp.dot(h, wa_ref[...], preferred_element_type=jnp.float32)
    yb_ref[...] = (jnp.dot(h, wb_ref[...], preferred_element_type=jnp.float32)
                   + b_ref[...])


def _stage_mid(p, r, wa, wb, b, width_out):
    return pl.pallas_call(
        _combine_body,
        grid=(N_ROW_BLKS,),
        in_specs=[_prows(2 * D_HID), _rows(2 * D_HID),
                  _full(wa.shape), _full(wb.shape), _full((1, width_out))],
        out_specs=[_rows(width_out), _rows(width_out)],
        out_shape=[jax.ShapeDtypeStruct((N_PACK, width_out), jnp.float32),
                   jax.ShapeDtypeStruct((N_PACK, width_out), jnp.float32)],
    )(p, r, wa, wb, b.reshape(1, -1))


def _combine_id_body(p_ref, r_ref, wb_ref, b_ref, h_ref, rb_ref):
    h = jnp.maximum(p_ref[0] + p_ref[1] + r_ref[...], 0.0)
    h_ref[...] = h
    rb_ref[...] = (jnp.dot(h, wb_ref[...], preferred_element_type=jnp.float32)
                   + b_ref[...])


def _stage_mid_id(p, r, wb, b, width_out):
    return pl.pallas_call(
        _combine_id_body,
        grid=(N_ROW_BLKS,),
        in_specs=[_prows(2 * D_HID), _rows(2 * D_HID),
                  _full(wb.shape), _full((1, width_out))],
        out_specs=[_rows(2 * D_HID), _rows(width_out)],
        out_shape=[jax.ShapeDtypeStruct((N_PACK, 2 * D_HID), jnp.float32),
                   jax.ShapeDtypeStruct((N_PACK, width_out), jnp.float32)],
    )(p, r, wb, b.reshape(1, -1))


def _head_body(p_ref, r3_ref, wrel3_ref, fc1w_ref, fc1b_ref,
               fc2w_ref, fc2b_ref, out_ref):
    agg = p_ref[0] + p_ref[1]
    z = jnp.dot(agg, wrel3_ref[...], preferred_element_type=jnp.float32)
    z = jnp.maximum(z + r3_ref[...], 0.0)
    z = jnp.dot(z, fc1w_ref[...], preferred_element_type=jnp.float32)
    z = jnp.maximum(z + fc1b_ref[...], 0.0)
    logits = (jnp.dot(z, fc2w_ref[...], preferred_element_type=jnp.float32)
              + fc2b_ref[...])
    n = logits.shape[-1] // 2
    out = []
    for l in (logits[:, :n], logits[:, n:]):
        m = jnp.max(l, axis=-1, keepdims=True)
        lse = jnp.log(jnp.sum(jnp.exp(l - m), axis=-1, keepdims=True)) + m
        out.append(l - lse)
    out_ref[...] = jnp.concatenate(out, axis=-1)


def _stage_head(p, r3, wrel3, fc1w, fc1b, fc2w, fc2b):
    n_cls2 = fc2w.shape[1]
    return pl.pallas_call(
        _head_body,
        grid=(N_ROW_BLKS,),
        in_specs=[_prows(2 * D_HID), _rows(256),
                  _full(wrel3.shape), _full(fc1w.shape), _full((1, 256)),
                  _full(fc2w.shape), _full((1, n_cls2))],
        out_specs=[_rows(n_cls2)],
        out_shape=[jax.ShapeDtypeStruct((N_PACK, n_cls2), jnp.float32)],
    )(p, r3, wrel3, fc1w, fc1b.reshape(1, -1),
      fc2w, fc2b.reshape(1, -1))[0]


def _bd(w):
    z = jnp.zeros(w.shape, w.dtype)
    return jnp.concatenate(
        [jnp.concatenate([w, z], axis=1), jnp.concatenate([z, w], axis=1)],
        axis=0)


def kernel(x, edge_index, W_rel1, W_root1, b1, W_rel2, W_root2, b2,
           W_rel3, W_root3, b3, fc1_W, fc1_b, fc2_W, fc2_b):
    src2d = edge_index[0].astype(jnp.int32).reshape(N_CHUNKS, CHUNK)
    dst2d = edge_index[1].astype(jnp.int32).reshape(N_CHUNKS, CHUNK)
    zeros = jnp.zeros((N_NODES, D_HID), jnp.float32)

    b2p = jnp.concatenate([b2, b2])
    b3p = jnp.concatenate([b3, b3])

    y1, r1 = _stage_in(x, W_rel1, W_root1,
                       jnp.concatenate([b1, b1]), 256, 128)
    p1 = _segsum_sc(y1.reshape(N_NODES, D_HID), src2d, dst2d, zeros)

    y2, r2 = _stage_mid(p1.reshape(NC, N_PACK, 2 * D_HID), r1,
                        _bd(W_rel2), _bd(W_root2), b2p, 128)
    p2 = _segsum_sc(y2.reshape(N_NODES, D_HID), src2d, dst2d, zeros)

    h2, r3 = _stage_mid_id(p2.reshape(NC, N_PACK, 2 * D_HID), r2,
                           _bd(W_root3), b3p, 256)
    p3 = _segsum_sc(h2.reshape(N_NODES, D_HID), src2d, dst2d, zeros)

    outp = _stage_head(p3.reshape(NC, N_PACK, 2 * D_HID), r3, _bd(W_rel3),
                       _bd(fc1_W), jnp.concatenate([fc1_b, fc1_b]),
                       _bd(fc2_W), jnp.concatenate([fc2_b, fc2_b]))
    n_cls = fc2_W.shape[1]
    return jnp.concatenate([outp[:, :n_cls], outp[:, n_cls:]], axis=0)


# speedup vs baseline: 1.0885x; 1.0364x over previous
import functools

import jax
import jax.numpy as jnp
from jax import lax
from jax.experimental import pallas as pl
from jax.experimental.pallas import tpu as pltpu
from jax.experimental.pallas import tpu_sc as plsc

N_NODES = 10000
N_EDGES = 160000
D_HID = 64

NC = 2
NS = 16
NW = NC * NS
CHUNK = 128
N_CHUNKS = N_EDGES // CHUNK
CH_PER_W = 40
GRP = 5
LAST_W = NW - 1
LAST_CH = N_CHUNKS - LAST_W * CH_PER_W
ACC_ROWS = N_NODES

ROWS_PER_SUB = 624
ROWS_REM = N_NODES - NS * ROWS_PER_SUB
REM_ROW0 = NS * ROWS_PER_SUB

N_PACK = N_NODES // 2
ROW_BLK = 1000
N_ROW_BLKS = N_PACK // ROW_BLK


def _segsum_sc(y, e3d, zeros):
    mesh = plsc.VectorSubcoreMesh(core_axis_name="c", subcore_axis_name="s")

    @functools.partial(
        pl.kernel,
        mesh=mesh,
        compiler_params=pltpu.CompilerParams(use_tc_tiling_on_sc=False),
        out_type=jax.ShapeDtypeStruct((NC, N_NODES, D_HID), jnp.float32),
        scratch_types=[
            pltpu.VMEM((CH_PER_W, CHUNK), jnp.int32),
            pltpu.VMEM((2, GRP, CHUNK), jnp.int32),
            pltpu.VMEM((GRP * CHUNK, D_HID), jnp.float32),
            pltpu.VMEM((GRP * CHUNK, D_HID), jnp.float32),
            pltpu.VMEM_SHARED((ACC_ROWS, D_HID), jnp.float32),
            pltpu.SemaphoreType.DMA,
            pltpu.SemaphoreType.DMA,
            pltpu.SemaphoreType.DMA,
        ],
    )
    def seg(y_hbm, e_hbm, zeros_hbm, out_hbm,
            srcv, dstv, bankA, bankB, acc, gsem, ssem, isem):
        cid = lax.axis_index("c")
        sid = lax.axis_index("s")
        wid = cid * NS + sid
        chunk0 = wid * CH_PER_W
        n_ch = jnp.where(wid == LAST_W, LAST_CH, CH_PER_W)

        def _perm(v):
            return jnp.where(v < N_PACK, 2 * v, 2 * v - (2 * N_PACK - 1))

        pltpu.sync_copy(e_hbm.at[0, pl.ds(chunk0, LAST_CH)],
                        srcv.at[pl.ds(0, LAST_CH)])

        @pl.when(wid < LAST_W)
        def _load_rest():
            pltpu.sync_copy(e_hbm.at[0, pl.ds(chunk0 + LAST_CH,
                                              CH_PER_W - LAST_CH)],
                            srcv.at[pl.ds(LAST_CH, CH_PER_W - LAST_CH)])

        def permute_src(i, _):
            r = lax.shift_right_logical(i, 3)
            c = lax.shift_left(jnp.bitwise_and(i, 7), 4)
            srcv[r, pl.ds(c, 16)] = _perm(srcv[r, pl.ds(c, 16)])
            return ()

        lax.fori_loop(0, n_ch * (CHUNK // 16), permute_src, ())

        def fire_i(db, grp):
            pltpu.async_copy(e_hbm.at[1, pl.ds(chunk0 + grp * GRP, GRP)],
                             dstv.at[db], isem)

        def drain_i(db, grp):
            pltpu.make_async_copy(
                e_hbm.at[1, pl.ds(chunk0 + grp * GRP, GRP)],
                dstv.at[db], isem).wait()

        def perm_d(db):
            for r in range(GRP):
                for c in range(CHUNK // 16):
                    sl = pl.ds(c * 16, 16)
                    dstv[db, r, sl] = _perm(dstv[db, r, sl])

        row0 = pl.multiple_of(sid * ROWS_PER_SUB, 8)
        pltpu.sync_copy(zeros_hbm.at[pl.ds(row0, ROWS_PER_SUB)],
                        acc.at[pl.ds(row0, ROWS_PER_SUB)])

        @pl.when(sid == NS - 1)
        def _zero_rem():
            pltpu.sync_copy(zeros_hbm.at[pl.ds(REM_ROW0, ROWS_REM)],
                            acc.at[pl.ds(REM_ROW0, ROWS_REM)])

        plsc.subcore_barrier()

        def fire_g(bank, grp):
            for j in range(GRP):
                pltpu.async_copy(y_hbm.at[srcv.at[grp * GRP + j]],
                                 bank.at[pl.ds(j * CHUNK, CHUNK)], gsem)

        def drain_g(bank, grp):
            for j in range(GRP):
                pltpu.make_async_copy(
                    y_hbm.at[srcv.at[grp * GRP + j]],
                    bank.at[pl.ds(j * CHUNK, CHUNK)], gsem).wait()

        def fire_s(bank, db):
            for j in range(GRP):
                pltpu.async_copy(bank.at[pl.ds(j * CHUNK, CHUNK)],
                                 acc.at[dstv.at[db, j]], ssem, add=True)

        def drain_s(bank, db):
            for j in range(GRP):
                pltpu.make_async_copy(
                    bank.at[pl.ds(j * CHUNK, CHUNK)],
                    acc.at[dstv.at[db, j]], ssem).wait()

        fire_i(0, 0)
        fire_g(bankA, 0)
        npair = n_ch // (2 * GRP)

        def pair(k, _):
            gA = k * 2
            drain_i(0, gA)
            perm_d(0)
            fire_i(1, gA + 1)
            drain_g(bankA, gA)
            fire_g(bankB, gA + 1)
            fire_s(bankA, 0)
            drain_s(bankA, 0)
            drain_i(1, gA + 1)
            perm_d(1)
            drain_g(bankB, gA + 1)

            @pl.when(k < npair - 1)
            def _next():
                fire_g(bankA, gA + 2)
                fire_i(0, gA + 2)

            fire_s(bankB, 1)
            drain_s(bankB, 1)
            return ()

        lax.fori_loop(0, npair, pair, ())

        plsc.subcore_barrier()
        pltpu.sync_copy(acc.at[pl.ds(row0, ROWS_PER_SUB)],
                        out_hbm.at[cid, pl.ds(row0, ROWS_PER_SUB)])

        @pl.when(sid == NS - 1)
        def _out_rem():
            pltpu.sync_copy(acc.at[pl.ds(REM_ROW0, ROWS_REM)],
                            out_hbm.at[cid, pl.ds(REM_ROW0, ROWS_REM)])

    return seg(y, e3d, zeros)


def _full(shape):
    return pl.BlockSpec(shape, lambda i: tuple(0 for _ in shape))


def _rows(width):
    return pl.BlockSpec((ROW_BLK, width), lambda i: (i, 0))


def _prows(width):
    return pl.BlockSpec((NC, ROW_BLK, width), lambda i: (0, i, 0))


def _mm2_body(xa_ref, xb_ref, wa_ref, wb_ref, b_ref, ya_ref, yb_ref):
    xa = xa_ref[...]
    xb = xb_ref[...]
    wa = wa_ref[...]
    wb = wb_ref[...]
    ya_ref[...] = jnp.concatenate(
        [jnp.dot(xa, wa, preferred_element_type=jnp.float32),
         jnp.dot(xb, wa, preferred_element_type=jnp.float32)], axis=1)
    yb_ref[...] = jnp.concatenate(
        [jnp.dot(xa, wb, preferred_element_type=jnp.float32),
         jnp.dot(xb, wb, preferred_element_type=jnp.float32)],
        axis=1) + b_ref[...]


def _stage_in(x, wa, wb, b2x, width_in, width_out):
    xlo = pl.BlockSpec((ROW_BLK, width_in), lambda i: (i, 0))
    xhi = pl.BlockSpec((ROW_BLK, width_in), lambda i: (i + N_ROW_BLKS, 0))
    return pl.pallas_call(
        _mm2_body,
        grid=(N_ROW_BLKS,),
        in_specs=[xlo, xhi, _full(wa.shape), _full(wb.shape),
                  _full((1, width_out))],
        out_specs=[_rows(width_out), _rows(width_out)],
        out_shape=[jax.ShapeDtypeStruct((N_PACK, width_out), jnp.float32),
                   jax.ShapeDtypeStruct((N_PACK, width_out), jnp.float32)],
    )(x, x, wa, wb, b2x.reshape(1, -1))


def _combine_body(p_ref, r_ref, wa_ref, wb_ref, b_ref, ya_ref, yb_ref):
    h = jnp.maximum(p_ref[0] + p_ref[1] + r_ref[...], 0.0)
    ya_ref[...] = jnp.dot(h, wa_ref[...], preferred_element_type=jnp.float32)
    yb_ref[...] = (jnp.dot(h, wb_ref[...], preferred_element_type=jnp.float32)
                   + b_ref[...])


def _stage_mid(p, r, wa, wb, b, width_out):
    return pl.pallas_call(
        _combine_body,
        grid=(N_ROW_BLKS,),
        in_specs=[_prows(2 * D_HID), _rows(2 * D_HID),
                  _full(wa.shape), _full(wb.shape), _full((1, width_out))],
        out_specs=[_rows(width_out), _rows(width_out)],
        out_shape=[jax.ShapeDtypeStruct((N_PACK, width_out), jnp.float32),
                   jax.ShapeDtypeStruct((N_PACK, width_out), jnp.float32)],
    )(p, r, wa, wb, b.reshape(1, -1))


def _combine_id_body(p_ref, r_ref, wb_ref, b_ref, h_ref, rb_ref):
    h = jnp.maximum(p_ref[0] + p_ref[1] + r_ref[...], 0.0)
    h_ref[...] = h
    rb_ref[...] = (jnp.dot(h, wb_ref[...], preferred_element_type=jnp.float32)
                   + b_ref[...])


def _stage_mid_id(p, r, wb, b, width_out):
    return pl.pallas_call(
        _combine_id_body,
        grid=(N_ROW_BLKS,),
        in_specs=[_prows(2 * D_HID), _rows(2 * D_HID),
                  _full(wb.shape), _full((1, width_out))],
        out_specs=[_rows(2 * D_HID), _rows(width_out)],
        out_shape=[jax.ShapeDtypeStruct((N_PACK, 2 * D_HID), jnp.float32),
                   jax.ShapeDtypeStruct((N_PACK, width_out), jnp.float32)],
    )(p, r, wb, b.reshape(1, -1))


def _head_body(p_ref, r3_ref, wrel3_ref, fc1w_ref, fc1b_ref,
               fc2w_ref, fc2b_ref, out_ref):
    agg = p_ref[0] + p_ref[1]
    z = jnp.dot(agg, wrel3_ref[...], preferred_element_type=jnp.float32)
    z = jnp.maximum(z + r3_ref[...], 0.0)
    z = jnp.dot(z, fc1w_ref[...], preferred_element_type=jnp.float32)
    z = jnp.maximum(z + fc1b_ref[...], 0.0)
    logits = (jnp.dot(z, fc2w_ref[...], preferred_element_type=jnp.float32)
              + fc2b_ref[...])
    n = logits.shape[-1] // 2
    out = []
    for l in (logits[:, :n], logits[:, n:]):
        m = jnp.max(l, axis=-1, keepdims=True)
        lse = jnp.log(jnp.sum(jnp.exp(l - m), axis=-1, keepdims=True)) + m
        out.append(l - lse)
    out_ref[...] = jnp.concatenate(out, axis=-1)


def _stage_head(p, r3, wrel3, fc1w, fc1b, fc2w, fc2b):
    n_cls2 = fc2w.shape[1]
    return pl.pallas_call(
        _head_body,
        grid=(N_ROW_BLKS,),
        in_specs=[_prows(2 * D_HID), _rows(256),
                  _full(wrel3.shape), _full(fc1w.shape), _full((1, 256)),
                  _full(fc2w.shape), _full((1, n_cls2))],
        out_specs=[_rows(n_cls2)],
        out_shape=[jax.ShapeDtypeStruct((N_PACK, n_cls2), jnp.float32)],
    )(p, r3, wrel3, fc1w, fc1b.reshape(1, -1),
      fc2w, fc2b.reshape(1, -1))[0]


def _bd(w):
    z = jnp.zeros(w.shape, w.dtype)
    return jnp.concatenate(
        [jnp.concatenate([w, z], axis=1), jnp.concatenate([z, w], axis=1)],
        axis=0)


def kernel(x, edge_index, W_rel1, W_root1, b1, W_rel2, W_root2, b2,
           W_rel3, W_root3, b3, fc1_W, fc1_b, fc2_W, fc2_b):
    e3d = edge_index.astype(jnp.int32).reshape(2, N_CHUNKS, CHUNK)
    zeros = jnp.zeros((N_NODES, D_HID), jnp.float32)

    b2p = jnp.concatenate([b2, b2])
    b3p = jnp.concatenate([b3, b3])

    y1, r1 = _stage_in(x, W_rel1, W_root1,
                       jnp.concatenate([b1, b1]), 256, 128)
    p1 = _segsum_sc(y1.reshape(N_NODES, D_HID), e3d, zeros)

    y2, r2 = _stage_mid(p1.reshape(NC, N_PACK, 2 * D_HID), r1,
                        _bd(W_rel2), _bd(W_root2), b2p, 128)
    p2 = _segsum_sc(y2.reshape(N_NODES, D_HID), e3d, zeros)

    h2, r3 = _stage_mid_id(p2.reshape(NC, N_PACK, 2 * D_HID), r2,
                           _bd(W_root3), b3p, 256)
    p3 = _segsum_sc(h2.reshape(N_NODES, D_HID), e3d, zeros)

    outp = _stage_head(p3.reshape(NC, N_PACK, 2 * D_HID), r3, _bd(W_rel3),
                       _bd(fc1_W), jnp.concatenate([fc1_b, fc1_b]),
                       _bd(fc2_W), jnp.concatenate([fc2_b, fc2_b]))
    n_cls = fc2_W.shape[1]
    return jnp.concatenate([outp[:, :n_cls], outp[:, n_cls:]], axis=0)
